# Initial kernel scaffold; baseline (speedup 1.0000x reference)
import functools

import jax
import jax.numpy as jnp
from jax import lax
from jax.experimental import pallas as pl
from jax.experimental.pallas import tpu as pltpu
from jax.experimental.pallas import tpu_sc as plsc

N = 10000
D = 128
NC = 2
NS = 16
NW = NC * NS
K = 128
N_PAD = N + 16


def _sc_aggregate(x, src, dst, zeros_feat, zeros_deg, ones_blk, e_pad):
    chunks_per_tile = e_pad // (NW * K)
    mesh = plsc.VectorSubcoreMesh(core_axis_name="c", subcore_axis_name="s")

    @functools.partial(
        pl.kernel,
        out_type=(
            jax.ShapeDtypeStruct((NC, N, D), jnp.float32),
            jax.ShapeDtypeStruct((NC, N, 16), jnp.float32),
        ),
        mesh=mesh,
        scratch_types=[
            pltpu.VMEM((K,), jnp.int32),
            pltpu.VMEM((K,), jnp.int32),
            pltpu.VMEM((K, D), jnp.float32),
            pltpu.VMEM((K, 16), jnp.float32),
            pltpu.VMEM_SHARED((N_PAD, D), jnp.float32),
            pltpu.VMEM_SHARED((N_PAD, 16), jnp.float32),
            pltpu.SemaphoreType.DMA,
        ],
    )
    def agg_kernel(x_hbm, src_hbm, dst_hbm, zf_hbm, zd_hbm, ones_hbm,
                   agg_out, deg_out,
                   src_v, dst_v, rows_v, ones_v, agg_sh, deg_sh, sem):
        cid = lax.axis_index("c")
        sid = lax.axis_index("s")

        zrows = N_PAD // NS
        z0 = sid * zrows
        pltpu.sync_copy(zf_hbm.at[pl.ds(z0, zrows)], agg_sh.at[pl.ds(z0, zrows)])
        pltpu.sync_copy(zd_hbm.at[pl.ds(z0, zrows)], deg_sh.at[pl.ds(z0, zrows)])
        pltpu.sync_copy(ones_hbm, ones_v)
        plsc.subcore_barrier()

        wid = cid * NS + sid
        tile_base = wid * chunks_per_tile * K

        def body(c, carry):
            base = tile_base + c * K
            pltpu.sync_copy(src_hbm.at[pl.ds(base, K)], src_v)
            pltpu.sync_copy(dst_hbm.at[pl.ds(base, K)], dst_v)
            pltpu.async_copy(x_hbm.at[src_v], rows_v, sem).wait()
            pltpu.sync_copy(rows_v, agg_sh.at[dst_v], add=True)
            pltpu.sync_copy(ones_v, deg_sh.at[dst_v], add=True)
            return carry

        lax.fori_loop(0, chunks_per_tile, body, 0)
        plsc.subcore_barrier()

        orows = N // NS
        o0 = sid * orows
        pltpu.sync_copy(agg_sh.at[pl.ds(o0, orows)],
                        agg_out.at[cid, pl.ds(o0, orows)])
        pltpu.sync_copy(deg_sh.at[pl.ds(o0, orows)],
                        deg_out.at[cid, pl.ds(o0, orows)])

    return agg_kernel(x, src, dst, zeros_feat, zeros_deg, ones_blk)


def _tc_dense(x, agg0, agg1, deg0, deg1, w_self, w_neigh, b2d):
    blk = 2000
    grid = (N // blk,)

    def body(x_ref, a0_ref, a1_ref, d0_ref, d1_ref, ws_ref, wn_ref, b_ref,
             o_ref):
        xb = x_ref[...]
        agg = a0_ref[...] + a1_ref[...]
        deg = d0_ref[...] + d1_ref[...]
        degc = jnp.clip(deg[:, 0:1], 1.0, None)
        out = (jnp.dot(xb, ws_ref[...], preferred_element_type=jnp.float32)
               + jnp.dot(agg / degc, wn_ref[...],
                         preferred_element_type=jnp.float32)
               + b_ref[...])
        o_ref[...] = jnp.maximum(jnp.maximum(out, 0.0), xb)

    row_spec = pl.BlockSpec((blk, D), lambda i: (i, 0))
    deg_spec = pl.BlockSpec((blk, 16), lambda i: (i, 0))
    full_spec = pl.BlockSpec((D, D), lambda i: (0, 0))
    bias_spec = pl.BlockSpec((1, D), lambda i: (0, 0))

    return pl.pallas_call(
        body,
        grid=grid,
        in_specs=[row_spec, row_spec, row_spec, deg_spec, deg_spec,
                  full_spec, full_spec, bias_spec],
        out_specs=row_spec,
        out_shape=jax.ShapeDtypeStruct((N, D), jnp.float32),
    )(x, agg0, agg1, deg0, deg1, w_self, w_neigh, b2d)


def kernel(x, edge_index, W_self, W_neigh, b):
    src = edge_index[0].astype(jnp.int32)
    dst = edge_index[1].astype(jnp.int32)
    e = src.shape[0]
    epp = NW * K
    e_pad = ((e + epp - 1) // epp) * epp
    pad = e_pad - e
    if pad:
        src = jnp.concatenate([src, jnp.zeros((pad,), jnp.int32)])
        dst = jnp.concatenate([dst, jnp.full((pad,), N, jnp.int32)])
    zeros_feat = jnp.zeros((N_PAD, D), jnp.float32)
    zeros_deg = jnp.zeros((N_PAD, 16), jnp.float32)
    ones_blk = jnp.ones((K, 16), jnp.float32)

    aggp, degp = _sc_aggregate(x, src, dst, zeros_feat, zeros_deg, ones_blk,
                               e_pad)
    return _tc_dense(x, aggp[0], aggp[1], degp[0], degp[1],
                     W_self, W_neigh, b.reshape(1, D))


# speedup vs baseline: 4.2652x; 4.2652x over previous
import functools

import jax
import jax.numpy as jnp
from jax import lax
from jax.experimental import pallas as pl
from jax.experimental.pallas import tpu as pltpu
from jax.experimental.pallas import tpu_sc as plsc

N = 10000
D = 128
NC = 2
NS = 16
NW = NC * NS
K = 128
N_PAD = 10240
RPT = N_PAD // NS
RCH = RPT // K


def _sc_aggregate(x, src, dst, zeros_f32, ones_f32, e_pad):
    cpt = e_pad // (NW * K)
    mesh = plsc.VectorSubcoreMesh(core_axis_name="c", subcore_axis_name="s")

    @functools.partial(
        pl.kernel,
        out_type=(
            jax.ShapeDtypeStruct((NC, N_PAD, D), jnp.float32),
            jax.ShapeDtypeStruct((NC, N_PAD, D), jnp.float32),
        ),
        mesh=mesh,
        scratch_types=[
            pltpu.VMEM((K,), jnp.int32),
            pltpu.VMEM((K,), jnp.int32),
            pltpu.VMEM((K, D), jnp.float32),
            pltpu.VMEM((K, D), jnp.float32),
            pltpu.VMEM_SHARED((N_PAD, D), jnp.float32),
            pltpu.SemaphoreType.DMA,
        ],
    )
    def agg_kernel(x_hbm, src_hbm, dst_hbm, zf_hbm, ones_hbm,
                   agg_out, deg_out,
                   src_v, dst_v, rows_v, ones_v, acc_sh, sem):
        cid = lax.axis_index("c")
        sid = lax.axis_index("s")
        wid = cid * NS + sid

        def zero_chunk(i, carry):
            r0 = sid * RPT + i * K
            pltpu.sync_copy(zf_hbm.at[pl.ds(i * K, K)], rows_v)
            pltpu.sync_copy(rows_v, acc_sh.at[pl.ds(r0, K)])
            return carry

        def dump_chunk(out_ref):
            def f(i, carry):
                r0 = sid * RPT + i * K
                pltpu.sync_copy(acc_sh.at[pl.ds(r0, K)], rows_v)
                pltpu.sync_copy(rows_v, out_ref.at[cid, pl.ds(r0, K)])
                return carry
            return f

        lax.fori_loop(0, RCH, zero_chunk, 0)
        pltpu.sync_copy(ones_hbm, ones_v)
        plsc.subcore_barrier()

        def agg_body(c, carry):
            base = (wid * cpt + c) * K
            pltpu.sync_copy(src_hbm.at[pl.ds(base, K)], src_v)
            pltpu.sync_copy(dst_hbm.at[pl.ds(base, K)], dst_v)
            pltpu.async_copy(x_hbm.at[src_v], rows_v, sem).wait()
            pltpu.sync_copy(rows_v, acc_sh.at[dst_v], add=True)
            return carry

        lax.fori_loop(0, cpt, agg_body, 0)
        plsc.subcore_barrier()
        lax.fori_loop(0, RCH, dump_chunk(agg_out), 0)
        plsc.subcore_barrier()

        lax.fori_loop(0, RCH, zero_chunk, 0)
        plsc.subcore_barrier()

        def deg_body(c, carry):
            base = (wid * cpt + c) * K
            pltpu.sync_copy(dst_hbm.at[pl.ds(base, K)], dst_v)
            pltpu.sync_copy(ones_v, acc_sh.at[dst_v], add=True)
            return carry

        lax.fori_loop(0, cpt, deg_body, 0)
        plsc.subcore_barrier()
        lax.fori_loop(0, RCH, dump_chunk(deg_out), 0)

    return agg_kernel(x, src, dst, zeros_f32, ones_f32)


def _tc_dense(x, agg0, agg1, deg0, deg1, w_self, w_neigh, b2d):
    blk = 2000
    grid = (N // blk,)

    def body(x_ref, a0_ref, a1_ref, d0_ref, d1_ref, ws_ref, wn_ref, b_ref,
             o_ref):
        xb = x_ref[...]
        agg = a0_ref[...] + a1_ref[...]
        deg = d0_ref[...] + d1_ref[...]
        degc = jnp.clip(deg[:, 0:1], 1.0, None)
        out = (jnp.dot(xb, ws_ref[...], preferred_element_type=jnp.float32)
               + jnp.dot(agg / degc, wn_ref[...],
                         preferred_element_type=jnp.float32)
               + b_ref[...])
        o_ref[...] = jnp.maximum(jnp.maximum(out, 0.0), xb)

    row_spec = pl.BlockSpec((blk, D), lambda i: (i, 0))
    full_spec = pl.BlockSpec((D, D), lambda i: (0, 0))
    bias_spec = pl.BlockSpec((1, D), lambda i: (0, 0))

    return pl.pallas_call(
        body,
        grid=grid,
        in_specs=[row_spec, row_spec, row_spec, row_spec, row_spec,
                  full_spec, full_spec, bias_spec],
        out_specs=row_spec,
        out_shape=jax.ShapeDtypeStruct((N, D), jnp.float32),
    )(x, agg0, agg1, deg0, deg1, w_self, w_neigh, b2d)


def kernel(x, edge_index, W_self, W_neigh, b):
    src = edge_index[0].astype(jnp.int32)
    dst = edge_index[1].astype(jnp.int32)
    e = src.shape[0]
    epp = NW * K
    e_pad = ((e + epp - 1) // epp) * epp
    pad = e_pad - e
    if pad:
        src = jnp.concatenate([src, jnp.zeros((pad,), jnp.int32)])
        dst = jnp.concatenate([dst, jnp.full((pad,), N, jnp.int32)])
    zeros_f32 = jnp.zeros((K * RCH, D), jnp.float32)
    ones_f32 = jnp.ones((K, D), jnp.float32)

    aggp, degp = _sc_aggregate(x, src, dst, zeros_f32, ones_f32, e_pad)
    return _tc_dense(x, aggp[0, :N], aggp[1, :N], degp[0, :N], degp[1, :N],
                     W_self, W_neigh, b.reshape(1, D))
